# baseline (speedup 1.0000x reference)
import functools

import jax
import jax.numpy as jnp
from jax import lax
from jax.experimental import pallas as pl
from jax.experimental.pallas import tpu as pltpu
from jax.experimental.pallas import tpu_sc as plsc

BATCH = 16384
EMBED = 32
NROWS = 1000000
NC = 2
NS = 16
NW = NC * NS
B_PER_W = BATCH // NW
CHUNK = 128
NCHUNK = B_PER_W // CHUNK
VR = B_PER_W // 16

MAIN = (NROWS // 128) * 128
BLOCKS = MAIN // 128
BPW = BLOCKS // 4
TPC = 24
FULL_CHUNKS = BPW // TPC
TAIL = NROWS - MAIN


def _detile_body(ut_hbm, it_hbm, utail_hbm, itail_hbm, uout_hbm, iout_hbm,
                 tiles_v, stages, tail_v, rsem0, rsem1, wsem0, wsem1):
  wid = lax.axis_index("s") * NC + lax.axis_index("c")
  tbl = wid // 16
  dh = (wid // 4) % 4
  r = wid % 4
  rsems = (rsem0, rsem1)
  wsems = (wsem0, wsem1)

  def do_table(src, tail_src, dst):
    def bb_of(c):
      return r * BPW + c * TPC

    def fire_reads(c, par):
      for t in range(TPC):
        pltpu.async_copy(
            src.at[dh, :, pl.ds((bb_of(c) + t) * 128, 128)],
            tiles_v.at[par, t], rsems[par])

    def drain(sem):
      pltpu.make_async_copy(dst.at[pl.ds(0, 8 * TPC * 128)],
                            stages[0], sem).wait()

    def restripe(par):
      def per_tile(t, carry):
        for dl in range(8):
          for c2 in range(8):
            stages[par][pl.ds(dl * (TPC * 128) + t * 128 + c2 * 16, 16)] = (
                tiles_v[par, t, dl, pl.ds(c2 * 16, 16)])
        return carry
      lax.fori_loop(0, TPC, per_tile, 0)

    def fire_writes(c, par):
      for dl in range(8):
        pltpu.async_copy(
            stages[par].at[pl.ds(dl * (TPC * 128), TPC * 128)],
            dst.at[pl.ds((dh * 8 + dl) * NROWS + bb_of(c) * 128, TPC * 128)],
            wsems[par])

    fire_reads(0, 0)

    def half_step(c, par):
      @pl.when(c + 1 < FULL_CHUNKS)
      def _():
        fire_reads(c + 1, 1 - par)
      drain(rsems[par])
      @pl.when(c >= 2)
      def _():
        drain(wsems[par])
      restripe(par)
      fire_writes(c, par)

    def chunk_loop(c, carry):
      @pl.when(lax.rem(c, 2) == 0)
      def _():
        half_step(c, 0)
      @pl.when(lax.rem(c, 2) == 1)
      def _():
        half_step(c, 1)
      return carry
    lax.fori_loop(0, FULL_CHUNKS, chunk_loop, 0)
    drain(wsems[0])
    drain(wsems[1])

    rem = BPW - FULL_CHUNKS * TPC
    for t in range(rem):
      pltpu.sync_copy(
          src.at[dh, :, pl.ds((bb_of(FULL_CHUNKS) + t) * 128, 128)],
          tiles_v.at[0, t])
    for t in range(rem):
      for dl in range(8):
        for c2 in range(8):
          stages[0][pl.ds(dl * (TPC * 128) + t * 128 + c2 * 16, 16)] = (
              tiles_v[0, t, dl, pl.ds(c2 * 16, 16)])
    for dl in range(8):
      pltpu.sync_copy(
          stages[0].at[pl.ds(dl * (TPC * 128), rem * 128)],
          dst.at[pl.ds((dh * 8 + dl) * NROWS + bb_of(FULL_CHUNKS) * 128,
                       rem * 128)])

    @pl.when(r == 0)
    def _():
      pltpu.sync_copy(tail_src, tail_v)
      for dl in range(8):
        d = dh * 8 + dl
        pltpu.sync_copy(
            tail_v.at[pl.ds(d * TAIL, TAIL)],
            dst.at[pl.ds(d * NROWS + MAIN, TAIL)])

  @pl.when(tbl == 0)
  def _():
    do_table(ut_hbm, utail_hbm, uout_hbm)

  @pl.when(tbl == 1)
  def _():
    do_table(it_hbm, itail_hbm, iout_hbm)


@functools.partial(
    pl.kernel,
    out_type=[jax.ShapeDtypeStruct((EMBED * NROWS,), jnp.float32),
              jax.ShapeDtypeStruct((EMBED * NROWS,), jnp.float32)],
    mesh=plsc.VectorSubcoreMesh(core_axis_name="c", subcore_axis_name="s",
                                num_cores=NC, num_subcores=NS),
    compiler_params=pltpu.CompilerParams(needs_layout_passes=False,
                                         use_tc_tiling_on_sc=True),
    scratch_types=[
        pltpu.VMEM((2, TPC, 8, 128), jnp.float32),
        pltpu.VMEM((8 * TPC * 128,), jnp.float32),
        pltpu.VMEM((8 * TPC * 128,), jnp.float32),
        pltpu.VMEM((EMBED * TAIL,), jnp.float32),
        pltpu.SemaphoreType.DMA,
        pltpu.SemaphoreType.DMA,
        pltpu.SemaphoreType.DMA,
        pltpu.SemaphoreType.DMA,
    ],
)
def _sc_detile(ut_hbm, it_hbm, utail_hbm, itail_hbm, uout_hbm, iout_hbm,
               tiles_v, stage0_v, stage1_v, tail_v, rsem0, rsem1, wsem0, wsem1):
  _detile_body(ut_hbm, it_hbm, utail_hbm, itail_hbm, uout_hbm, iout_hbm,
               tiles_v, (stage0_v, stage1_v), tail_v, rsem0, rsem1, wsem0, wsem1)


def _dot_body(uidx_hbm, iidx_hbm, utab_hbm, itab_hbm, out_hbm,
              uraw_v, iraw_v, upidx_v, vpidx_v, ucol_v, vcol_v, acc_v,
              sem0, sem1):
  wid = lax.axis_index("s") * NC + lax.axis_index("c")
  base = wid * B_PER_W

  pltpu.sync_copy(uidx_hbm.at[pl.ds(base, B_PER_W)], uraw_v)
  pltpu.sync_copy(iidx_hbm.at[pl.ds(base, B_PER_W)], iraw_v)

  for i in range(VR):
    acc_v[pl.ds(i * 16, 16)] = jnp.zeros((16,), jnp.float32)

  sems = (sem0, sem1)

  def build_indices(d, par):
    off = d * NROWS
    for i in range(VR):
      j, sl = i // 8, (i % 8) * 16
      upidx_v[par, j, pl.ds(sl, 16)] = uraw_v[pl.ds(i * 16, 16)] + off
      vpidx_v[par, j, pl.ds(sl, 16)] = iraw_v[pl.ds(i * 16, 16)] + off

  def fire(par):
    copies = []
    for j in range(NCHUNK):
      copies.append(pltpu.async_copy(
          utab_hbm.at[upidx_v.at[par, j]],
          ucol_v.at[par, pl.ds(j * CHUNK, CHUNK)], sems[par]))
      copies.append(pltpu.async_copy(
          itab_hbm.at[vpidx_v.at[par, j]],
          vcol_v.at[par, pl.ds(j * CHUNK, CHUNK)], sems[par]))
    return copies

  build_indices(0, 0)
  inflight = fire(0)
  for d in range(EMBED):
    par = d % 2
    nxt = (d + 1) % 2
    if d + 1 < EMBED:
      build_indices(d + 1, nxt)
      nxt_copies = fire(nxt)
    for c in inflight:
      c.wait()
    for i in range(VR):
      acc_v[pl.ds(i * 16, 16)] = (
          acc_v[pl.ds(i * 16, 16)]
          + ucol_v[par, pl.ds(i * 16, 16)] * vcol_v[par, pl.ds(i * 16, 16)])
    if d + 1 < EMBED:
      inflight = nxt_copies

  pltpu.sync_copy(acc_v, out_hbm.at[pl.ds(base, B_PER_W)])


@functools.partial(
    pl.kernel,
    out_type=jax.ShapeDtypeStruct((BATCH,), jnp.float32),
    mesh=plsc.VectorSubcoreMesh(core_axis_name="c", subcore_axis_name="s",
                                num_cores=NC, num_subcores=NS),
    compiler_params=pltpu.CompilerParams(needs_layout_passes=False),
    scratch_types=[
        pltpu.VMEM((B_PER_W,), jnp.int32),
        pltpu.VMEM((B_PER_W,), jnp.int32),
        pltpu.VMEM((2, NCHUNK, CHUNK), jnp.int32),
        pltpu.VMEM((2, NCHUNK, CHUNK), jnp.int32),
        pltpu.VMEM((2, B_PER_W), jnp.float32),
        pltpu.VMEM((2, B_PER_W), jnp.float32),
        pltpu.VMEM((B_PER_W,), jnp.float32),
        pltpu.SemaphoreType.DMA,
        pltpu.SemaphoreType.DMA,
    ],
)
def _sc_dot(uidx_hbm, iidx_hbm, utab_hbm, itab_hbm, out_hbm,
            uraw_v, iraw_v, upidx_v, vpidx_v, ucol_v, vcol_v, acc_v,
            sem0, sem1):
  _dot_body(uidx_hbm, iidx_hbm, utab_hbm, itab_hbm, out_hbm,
            uraw_v, iraw_v, upidx_v, vpidx_v, ucol_v, vcol_v, acc_v,
            sem0, sem1)


def kernel(inputs, user_table, item_table):
  uidx = inputs[:, 0]
  iidx = inputs[:, 1]
  ut_view = user_table.T.reshape(4, 8, NROWS)
  it_view = item_table.T.reshape(4, 8, NROWS)
  utail = user_table[MAIN:].T.reshape(-1)
  itail = item_table[MAIN:].T.reshape(-1)
  uflat, iflat = _sc_detile(ut_view, it_view, utail, itail)
  return _sc_dot(uidx, iidx, uflat, iflat)


# speedup vs baseline: 1.0190x; 1.0190x over previous
import functools

import jax
import jax.numpy as jnp
from jax import lax
from jax.experimental import pallas as pl
from jax.experimental.pallas import tpu as pltpu
from jax.experimental.pallas import tpu_sc as plsc

BATCH = 16384
EMBED = 32
NROWS = 1000000
NC = 2
NS = 16
NW = NC * NS
B_PER_W = BATCH // NW
CHUNK = 128
NCHUNK = B_PER_W // CHUNK
VR = B_PER_W // 16

MAIN = (NROWS // 128) * 128
BLOCKS = MAIN // 128
BPW = BLOCKS // 4
TPC = 16
FULL_CHUNKS = BPW // TPC
TAIL = NROWS - MAIN


def _detile_body(ut_hbm, it_hbm, utail_hbm, itail_hbm, uout_hbm, iout_hbm,
                 tiles_v, stages, tail_v, rsem0, rsem1, wsem0, wsem1):
  wid = lax.axis_index("s") * NC + lax.axis_index("c")
  tbl = wid // 16
  dh = (wid // 4) % 4
  r = wid % 4
  rsems = (rsem0, rsem1)
  wsems = (wsem0, wsem1)

  def do_table(src, tail_src, dst):
    def bb_of(c):
      return r * BPW + c * TPC

    def fire_reads(c, par):
      for t in range(TPC):
        pltpu.async_copy(
            src.at[dh, :, pl.ds((bb_of(c) + t) * 128, 128)],
            tiles_v.at[par, t], rsems[par])

    def drain(sem):
      pltpu.make_async_copy(dst.at[pl.ds(0, 8 * TPC * 128)],
                            stages[0], sem).wait()

    def restripe(par):
      def per_tile(t, carry):
        for dl in range(8):
          for c2 in range(8):
            stages[par][pl.ds(dl * (TPC * 128) + t * 128 + c2 * 16, 16)] = (
                tiles_v[par, t, dl, pl.ds(c2 * 16, 16)])
        return carry
      lax.fori_loop(0, TPC, per_tile, 0)

    def fire_writes(c, par):
      for dl in range(8):
        pltpu.async_copy(
            stages[par].at[pl.ds(dl * (TPC * 128), TPC * 128)],
            dst.at[pl.ds((dh * 8 + dl) * NROWS + bb_of(c) * 128, TPC * 128)],
            wsems[par])

    fire_reads(0, 0)

    def half_step(c, par):
      @pl.when(c + 1 < FULL_CHUNKS)
      def _():
        fire_reads(c + 1, 1 - par)
      drain(rsems[par])
      @pl.when(c >= 2)
      def _():
        drain(wsems[par])
      restripe(par)
      fire_writes(c, par)

    def chunk_loop(c, carry):
      @pl.when(lax.rem(c, 2) == 0)
      def _():
        half_step(c, 0)
      @pl.when(lax.rem(c, 2) == 1)
      def _():
        half_step(c, 1)
      return carry
    lax.fori_loop(0, FULL_CHUNKS, chunk_loop, 0)
    drain(wsems[0])
    drain(wsems[1])

    rem = BPW - FULL_CHUNKS * TPC
    for t in range(rem):
      pltpu.sync_copy(
          src.at[dh, :, pl.ds((bb_of(FULL_CHUNKS) + t) * 128, 128)],
          tiles_v.at[0, t])
    for t in range(rem):
      for dl in range(8):
        for c2 in range(8):
          stages[0][pl.ds(dl * (TPC * 128) + t * 128 + c2 * 16, 16)] = (
              tiles_v[0, t, dl, pl.ds(c2 * 16, 16)])
    for dl in range(8):
      pltpu.sync_copy(
          stages[0].at[pl.ds(dl * (TPC * 128), rem * 128)],
          dst.at[pl.ds((dh * 8 + dl) * NROWS + bb_of(FULL_CHUNKS) * 128,
                       rem * 128)])

    @pl.when(r == 0)
    def _():
      pltpu.sync_copy(tail_src, tail_v)
      for dl in range(8):
        d = dh * 8 + dl
        pltpu.sync_copy(
            tail_v.at[pl.ds(d * TAIL, TAIL)],
            dst.at[pl.ds(d * NROWS + MAIN, TAIL)])

  @pl.when(tbl == 0)
  def _():
    do_table(ut_hbm, utail_hbm, uout_hbm)

  @pl.when(tbl == 1)
  def _():
    do_table(it_hbm, itail_hbm, iout_hbm)


@functools.partial(
    pl.kernel,
    out_type=[jax.ShapeDtypeStruct((EMBED * NROWS,), jnp.float32),
              jax.ShapeDtypeStruct((EMBED * NROWS,), jnp.float32)],
    mesh=plsc.VectorSubcoreMesh(core_axis_name="c", subcore_axis_name="s",
                                num_cores=NC, num_subcores=NS),
    compiler_params=pltpu.CompilerParams(needs_layout_passes=False,
                                         use_tc_tiling_on_sc=True),
    scratch_types=[
        pltpu.VMEM((2, TPC, 8, 128), jnp.float32),
        pltpu.VMEM((8 * TPC * 128,), jnp.float32),
        pltpu.VMEM((8 * TPC * 128,), jnp.float32),
        pltpu.VMEM((EMBED * TAIL,), jnp.float32),
        pltpu.SemaphoreType.DMA,
        pltpu.SemaphoreType.DMA,
        pltpu.SemaphoreType.DMA,
        pltpu.SemaphoreType.DMA,
    ],
)
def _sc_detile(ut_hbm, it_hbm, utail_hbm, itail_hbm, uout_hbm, iout_hbm,
               tiles_v, stage0_v, stage1_v, tail_v, rsem0, rsem1, wsem0, wsem1):
  _detile_body(ut_hbm, it_hbm, utail_hbm, itail_hbm, uout_hbm, iout_hbm,
               tiles_v, (stage0_v, stage1_v), tail_v, rsem0, rsem1, wsem0, wsem1)


def _dot_body(uidx_hbm, iidx_hbm, utab_hbm, itab_hbm, out_hbm,
              uraw_v, iraw_v, upidx_v, vpidx_v, ucol_v, vcol_v, acc_v,
              sem0, sem1):
  wid = lax.axis_index("s") * NC + lax.axis_index("c")
  base = wid * B_PER_W

  pltpu.sync_copy(uidx_hbm.at[pl.ds(base, B_PER_W)], uraw_v)
  pltpu.sync_copy(iidx_hbm.at[pl.ds(base, B_PER_W)], iraw_v)

  for i in range(VR):
    acc_v[pl.ds(i * 16, 16)] = jnp.zeros((16,), jnp.float32)

  sems = (sem0, sem1)

  def build_indices(d, par):
    off = d * NROWS
    for i in range(VR):
      j, sl = i // 8, (i % 8) * 16
      upidx_v[par, j, pl.ds(sl, 16)] = uraw_v[pl.ds(i * 16, 16)] + off
      vpidx_v[par, j, pl.ds(sl, 16)] = iraw_v[pl.ds(i * 16, 16)] + off

  def fire(par):
    copies = []
    for j in range(NCHUNK):
      copies.append(pltpu.async_copy(
          utab_hbm.at[upidx_v.at[par, j]],
          ucol_v.at[par, pl.ds(j * CHUNK, CHUNK)], sems[par]))
      copies.append(pltpu.async_copy(
          itab_hbm.at[vpidx_v.at[par, j]],
          vcol_v.at[par, pl.ds(j * CHUNK, CHUNK)], sems[par]))
    return copies

  build_indices(0, 0)
  inflight = fire(0)
  for d in range(EMBED):
    par = d % 2
    nxt = (d + 1) % 2
    if d + 1 < EMBED:
      build_indices(d + 1, nxt)
      nxt_copies = fire(nxt)
    for c in inflight:
      c.wait()
    for i in range(VR):
      acc_v[pl.ds(i * 16, 16)] = (
          acc_v[pl.ds(i * 16, 16)]
          + ucol_v[par, pl.ds(i * 16, 16)] * vcol_v[par, pl.ds(i * 16, 16)])
    if d + 1 < EMBED:
      inflight = nxt_copies

  pltpu.sync_copy(acc_v, out_hbm.at[pl.ds(base, B_PER_W)])


@functools.partial(
    pl.kernel,
    out_type=jax.ShapeDtypeStruct((BATCH,), jnp.float32),
    mesh=plsc.VectorSubcoreMesh(core_axis_name="c", subcore_axis_name="s",
                                num_cores=NC, num_subcores=NS),
    compiler_params=pltpu.CompilerParams(needs_layout_passes=False),
    scratch_types=[
        pltpu.VMEM((B_PER_W,), jnp.int32),
        pltpu.VMEM((B_PER_W,), jnp.int32),
        pltpu.VMEM((2, NCHUNK, CHUNK), jnp.int32),
        pltpu.VMEM((2, NCHUNK, CHUNK), jnp.int32),
        pltpu.VMEM((2, B_PER_W), jnp.float32),
        pltpu.VMEM((2, B_PER_W), jnp.float32),
        pltpu.VMEM((B_PER_W,), jnp.float32),
        pltpu.SemaphoreType.DMA,
        pltpu.SemaphoreType.DMA,
    ],
)
def _sc_dot(uidx_hbm, iidx_hbm, utab_hbm, itab_hbm, out_hbm,
            uraw_v, iraw_v, upidx_v, vpidx_v, ucol_v, vcol_v, acc_v,
            sem0, sem1):
  _dot_body(uidx_hbm, iidx_hbm, utab_hbm, itab_hbm, out_hbm,
            uraw_v, iraw_v, upidx_v, vpidx_v, ucol_v, vcol_v, acc_v,
            sem0, sem1)


def kernel(inputs, user_table, item_table):
  uidx = inputs[:, 0]
  iidx = inputs[:, 1]
  ut_view = user_table.T.reshape(4, 8, NROWS)
  it_view = item_table.T.reshape(4, 8, NROWS)
  utail = user_table[MAIN:].T.reshape(-1)
  itail = item_table[MAIN:].T.reshape(-1)
  uflat, iflat = _sc_detile(ut_view, it_view, utail, itail)
  return _sc_dot(uidx, iidx, uflat, iflat)
